# baseline (speedup 1.0000x reference)
import functools

import jax
import jax.numpy as jnp
from jax import lax
from jax.experimental import pallas as pl
from jax.experimental.pallas import tpu as pltpu
from jax.experimental.pallas import tpu_sc as plsc

N = 10000
E = 320000
D = 128
DE = 16
T_STEPS = 3
NC, NS, L = 2, 16, 16
NW = NC * NS
EW = E // NW
K = 100
CH = EW // K
NP = 10240
RPT = NP // NS
EB = 4000
KE = 50
CHE = EW // KE


def _sc_mesh():
    return plsc.VectorSubcoreMesh(core_axis_name="c", subcore_axis_name="s",
                                  num_cores=NC, num_subcores=NS)


def _deg_body(dst_hbm, zeros_hbm, ones_hbm, out_hbm, idx_v, ones_v, deg_sp, sem):
    del sem
    cid = lax.axis_index("c")
    sid = lax.axis_index("s")
    wid = cid * NS + sid
    pltpu.sync_copy(zeros_hbm, deg_sp.at[pl.ds(sid * RPT, RPT)])
    pltpu.sync_copy(ones_hbm, ones_v)
    pltpu.sync_copy(dst_hbm.at[wid], idx_v)
    plsc.subcore_barrier()

    def chunk(i, carry):
        pltpu.sync_copy(ones_v, deg_sp.at[idx_v.at[i]], add=True)
        return carry

    lax.fori_loop(0, CH, chunk, 0)
    plsc.subcore_barrier()
    pltpu.sync_copy(deg_sp.at[pl.ds(sid * RPT, RPT)],
                    out_hbm.at[cid, pl.ds(sid * RPT, RPT)])


def _deg_call(dst3, zeros16, ones16, interpret=False):
    f = pl.kernel(
        _deg_body,
        out_type=jax.ShapeDtypeStruct((NC, NP), jnp.float32),
        mesh=_sc_mesh(),
        scratch_types=[
            pltpu.VMEM((CH, K), jnp.int32),
            pltpu.VMEM((K,), jnp.float32),
            pltpu.VMEM_SHARED((NP,), jnp.float32),
            pltpu.SemaphoreType.DMA,
        ],
        interpret=interpret,
    )
    return f(dst3, zeros16, ones16)


def _prop_body(sd_hbm, xws_hbm, zeros_hbm, out_hbm,
               idx_v, rows0, acc_sp, gsem0):
    cid = lax.axis_index("c")
    sid = lax.axis_index("s")
    wid = cid * NS + sid
    pltpu.sync_copy(zeros_hbm, acc_sp.at[pl.ds(sid * RPT, RPT)])
    pltpu.sync_copy(sd_hbm.at[wid], idx_v)
    plsc.subcore_barrier()

    def chunk(i, carry):
        pltpu.async_copy(xws_hbm.at[idx_v.at[i]], rows0, gsem0).wait()
        pltpu.sync_copy(rows0, acc_sp.at[idx_v.at[CH + i]], add=True)
        return carry

    lax.fori_loop(0, CH, chunk, 0)
    plsc.subcore_barrier()
    pltpu.sync_copy(acc_sp.at[pl.ds(sid * RPT, RPT)],
                    out_hbm.at[cid, pl.ds(sid * RPT, RPT)])


def _prop_call(sd4, xws, zerosD, interpret=False):
    f = pl.kernel(
        _prop_body,
        out_type=jax.ShapeDtypeStruct((NC, NP, D), jnp.float32),
        mesh=_sc_mesh(),
        scratch_types=[
            pltpu.VMEM((2 * CH, K), jnp.int32),
            pltpu.VMEM((K, D), jnp.float32),
            pltpu.VMEM_SHARED((NP, D), jnp.float32),
            pltpu.SemaphoreType.DMA,
        ],
        interpret=interpret,
    )
    return f(sd4, xws, zerosD)


def _edge_body(src_hbm, dst_hbm, a_hbm, b_hbm, c_hbm, w2_hbm, out_hbm,
               src_v, dst_v, a0, b0, c0, z0, a1, b1, c1, z1, w2_v,
               gsem0, gsem1, zsem0, zsem1):
    cid = lax.axis_index("c")
    sid = lax.axis_index("s")
    wid = cid * NS + sid
    pltpu.sync_copy(src_hbm.at[wid], src_v)
    pltpu.sync_copy(dst_hbm.at[wid], dst_v)
    pltpu.sync_copy(w2_hbm, w2_v)
    w2s = [w2_v[pl.ds(16 * d, 16)] for d in range(D // 16)]
    bufs = ((a0, b0, c0, z0, gsem0, zsem0), (a1, b1, c1, z1, gsem1, zsem1))

    def issue(i, a_v, b_v, c_v, gsem):
        pltpu.async_copy(a_hbm.at[src_v.at[i]], a_v, gsem)
        pltpu.async_copy(b_hbm.at[dst_v.at[i]], b_v, gsem)
        pltpu.async_copy(c_hbm.at[wid, i], c_v, gsem)

    issue(0, a0, b0, c0, gsem0)
    issue(1, a1, b1, c1, gsem1)

    def pair(p, carry):
        for bsel in range(2):
            a_v, b_v, c_v, z_v, gsem, zsem = bufs[bsel]
            i = 2 * p + bsel
            for dbuf in (a_v, b_v, c_v):
                pltpu.make_async_copy(c_hbm.at[wid, 0], dbuf, gsem).wait()
            @pl.when(p > 0)
            def _():
                pltpu.make_async_copy(z_v, out_hbm.at[wid, 0], zsem).wait()

            @plsc.parallel_loop(0, KE, unroll=8)
            def edge_fn(e):
                s_acc = jnp.zeros((L,), jnp.float32)
                for d in range(D // 16):
                    va = a_v[e, pl.ds(16 * d, 16)]
                    vb = b_v[e, pl.ds(16 * d, 16)]
                    vc = c_v[e, pl.ds(16 * d, 16)]
                    s = jnp.maximum(va + vb + vc, 0.0)
                    s_acc = s_acc + s * w2s[d]
                z_v[e, :] = s_acc

            pltpu.async_copy(z_v, out_hbm.at[wid, i], zsem)

            @pl.when(i + 2 < CHE)
            def _():
                issue(i + 2, a_v, b_v, c_v, gsem)
        return carry

    lax.fori_loop(0, CHE // 2, pair, 0)
    for a_v, b_v, c_v, z_v, gsem, zsem in bufs:
        pltpu.make_async_copy(z_v, out_hbm.at[wid, 0], zsem).wait()


def _edge_call(src3, dst3, A, B, C4, w2, interpret=False):
    f = pl.kernel(
        _edge_body,
        out_type=jax.ShapeDtypeStruct((NW, CHE, KE, L), jnp.float32),
        mesh=_sc_mesh(),
        scratch_types=[
            pltpu.VMEM((CHE, KE), jnp.int32),
            pltpu.VMEM((CHE, KE), jnp.int32),
            pltpu.VMEM((KE, D), jnp.float32),
            pltpu.VMEM((KE, D), jnp.float32),
            pltpu.VMEM((KE, D), jnp.float32),
            pltpu.VMEM((KE, L), jnp.float32),
            pltpu.VMEM((KE, D), jnp.float32),
            pltpu.VMEM((KE, D), jnp.float32),
            pltpu.VMEM((KE, D), jnp.float32),
            pltpu.VMEM((KE, L), jnp.float32),
            pltpu.VMEM((D,), jnp.float32),
            pltpu.SemaphoreType.DMA,
            pltpu.SemaphoreType.DMA,
            pltpu.SemaphoreType.DMA,
            pltpu.SemaphoreType.DMA,
        ],
        interpret=interpret,
    )
    return f(src3, dst3, A, B, C4, w2)


def _fin_body(z_ref, sel_ref, b2_ref, out_ref):
    out_ref[...] = lax.dot_general(
        z_ref[...], sel_ref[...], (((1,), (0,)), ((), ())),
        preferred_element_type=jnp.float32) + b2_ref[0]


def _fin_call(z2, sel, b2, interpret=False):
    f = pl.pallas_call(
        _fin_body,
        out_shape=jax.ShapeDtypeStruct((E // 32, 32), jnp.float32),
        interpret=interpret,
    )
    return f(z2, sel, b2)


def _lstm_gates(wt, h, W_ih, W_hh, b_ih, b_hh):
    dn = (((1,), (1,)), ((), ()))
    gates = (lax.dot_general(wt, W_ih, dn, preferred_element_type=jnp.float32)
             + lax.dot_general(h, W_hh, dn, preferred_element_type=jnp.float32)
             + b_ih[None, :] + b_hh[None, :])
    return gates


def _prep_body(x_ref, w0t_ref, wih_ref, whh_ref, bih_ref, bhh_ref, degp_ref,
               xw_ref, xws_ref, dis_ref):
    wt = w0t_ref[...]
    h = jnp.zeros((D, D), jnp.float32)
    c = jnp.zeros((D, D), jnp.float32)
    W_ih = wih_ref[...]
    W_hh = whh_ref[...]
    b_ih = bih_ref[...]
    b_hh = bhh_ref[...]
    for _ in range(T_STEPS):
        gates = _lstm_gates(wt, h, W_ih, W_hh, b_ih, b_hh)
        gi = jax.nn.sigmoid(gates[:, 0 * D:1 * D])
        gf = jax.nn.sigmoid(gates[:, 1 * D:2 * D])
        gg = jnp.tanh(gates[:, 2 * D:3 * D])
        go = jax.nn.sigmoid(gates[:, 3 * D:4 * D])
        c = gf * c + gi * gg
        h = go * jnp.tanh(c)
        wt = h
    xw = lax.dot_general(x_ref[...], h, (((1,), (1,)), ((), ())),
                         preferred_element_type=jnp.float32)
    deg = degp_ref[0, :N] + degp_ref[1, :N] + 1.0
    dis = lax.rsqrt(deg)
    xw_ref[...] = xw
    xws_ref[...] = xw * dis[:, None]
    dis_ref[...] = dis


def _prep_call(x, w0t, wih, whh, bih, bhh, degp, interpret=False):
    f = pl.pallas_call(
        _prep_body,
        out_shape=[
            jax.ShapeDtypeStruct((N, D), jnp.float32),
            jax.ShapeDtypeStruct((N, D), jnp.float32),
            jax.ShapeDtypeStruct((N,), jnp.float32),
        ],
        interpret=interpret,
    )
    return f(x, w0t, wih, whh, bih, bhh, degp)


def _node_body(pacc_ref, xw_ref, dis_ref, w1a_ref, w1b_ref, b1_ref,
               a_ref, b_ref):
    dis = dis_ref[...]
    s = ((pacc_ref[0, :N] + pacc_ref[1, :N]) * dis[:, None]
         + (dis * dis)[:, None] * xw_ref[...])
    emb = jnp.maximum(s, 0.0)
    dn = (((1,), (1,)), ((), ()))
    a_ref[...] = lax.dot_general(emb, w1a_ref[...], dn,
                                 preferred_element_type=jnp.float32)
    b_ref[...] = lax.dot_general(emb, w1b_ref[...], dn,
                                 preferred_element_type=jnp.float32) + b1_ref[...][None, :]


def _node_call(pacc, xw, dis, w1a, w1b, b1, interpret=False):
    f = pl.pallas_call(
        _node_body,
        out_shape=[
            jax.ShapeDtypeStruct((N, D), jnp.float32),
            jax.ShapeDtypeStruct((N, D), jnp.float32),
        ],
        interpret=interpret,
    )
    return f(pacc, xw, dis, w1a, w1b, b1)


def _edgec_body(attr_ref, w1c_ref, c_ref):
    c_ref[...] = lax.dot_general(attr_ref[...], w1c_ref[...],
                                 (((1,), (1,)), ((), ())),
                                 preferred_element_type=jnp.float32)


def _edgec_call(edge_attr, w1c, interpret=False):
    f = pl.pallas_call(
        _edgec_body,
        grid=(E // EB,),
        in_specs=[
            pl.BlockSpec((EB, DE), lambda i: (i, 0)),
            pl.BlockSpec((D, DE), lambda i: (0, 0)),
        ],
        out_specs=pl.BlockSpec((EB, D), lambda i: (i, 0)),
        out_shape=jax.ShapeDtypeStruct((E, D), jnp.float32),
        interpret=interpret,
    )
    return f(edge_attr, w1c)


def _run(x, edge_index, edge_attr, initial_weights, lstm_W_ih, lstm_W_hh,
         lstm_b_ih, lstm_b_hh, mlp_W1, mlp_b1, mlp_W2, mlp_b2,
         interpret=False):
    src = edge_index[0]
    dst = edge_index[1]
    src3 = src.reshape(NW, CH, K)
    dst3 = dst.reshape(NW, CH, K)

    zeros1 = jnp.zeros((RPT,), jnp.float32)
    ones1 = jnp.ones((K,), jnp.float32)
    degp = _deg_call(dst3, zeros1, ones1, interpret=interpret)

    w1a = mlp_W1[:, :D]
    w1b = mlp_W1[:, D:2 * D]
    w1c = mlp_W1[:, 2 * D:]
    C = _edgec_call(edge_attr, w1c, interpret=interpret)
    C4 = C.reshape(NW, CHE, KE, D)

    xw, xws, dis = _prep_call(x, initial_weights.T, lstm_W_ih, lstm_W_hh,
                              lstm_b_ih, lstm_b_hh, degp, interpret=interpret)

    zerosD = jnp.zeros((RPT, D), jnp.float32)
    sd4 = jnp.concatenate([src3, dst3], axis=1)
    pacc = _prop_call(sd4, xws, zerosD, interpret=interpret)

    A, B = _node_call(pacc, xw, dis, w1a, w1b, mlp_b1, interpret=interpret)

    w2 = mlp_W2[0]
    srcE = src.reshape(NW, CHE, KE)
    dstE = dst.reshape(NW, CHE, KE)
    z = _edge_call(srcE, dstE, A, B, C4, w2, interpret=interpret)
    sel = jnp.repeat(jnp.eye(32, dtype=jnp.float32), L, axis=0)
    out2 = _fin_call(z.reshape(E // 32, 32 * L), sel, mlp_b2,
                     interpret=interpret)
    return out2.reshape(E)


def kernel(x, edge_index, edge_attr, initial_weights, lstm_W_ih, lstm_W_hh,
           lstm_b_ih, lstm_b_hh, mlp_W1, mlp_b1, mlp_W2, mlp_b2):
    return _run(x, edge_index, edge_attr, initial_weights, lstm_W_ih,
                lstm_W_hh, lstm_b_ih, lstm_b_hh, mlp_W1, mlp_b1, mlp_W2,
                mlp_b2)


# speedup vs baseline: 1.0237x; 1.0237x over previous
import functools

import jax
import jax.numpy as jnp
from jax import lax
from jax.experimental import pallas as pl
from jax.experimental.pallas import tpu as pltpu
from jax.experimental.pallas import tpu_sc as plsc

N = 10000
E = 320000
D = 128
DE = 16
T_STEPS = 3
NC, NS, L = 2, 16, 16
NW = NC * NS
EW = E // NW
K = 100
CH = EW // K
NP = 10240
RPT = NP // NS
EB = 4000
KE = 50
CHE = EW // KE


def _sc_mesh():
    return plsc.VectorSubcoreMesh(core_axis_name="c", subcore_axis_name="s",
                                  num_cores=NC, num_subcores=NS)


def _deg_body(dst_hbm, zeros_hbm, ones_hbm, out_hbm, idx_v, ones_v, deg_sp, sem):
    del sem
    cid = lax.axis_index("c")
    sid = lax.axis_index("s")
    wid = cid * NS + sid
    pltpu.sync_copy(zeros_hbm, deg_sp.at[pl.ds(sid * RPT, RPT)])
    pltpu.sync_copy(ones_hbm, ones_v)
    pltpu.sync_copy(dst_hbm.at[wid], idx_v)
    plsc.subcore_barrier()

    def chunk(i, carry):
        pltpu.sync_copy(ones_v, deg_sp.at[idx_v.at[i]], add=True)
        return carry

    lax.fori_loop(0, CH, chunk, 0)
    plsc.subcore_barrier()
    pltpu.sync_copy(deg_sp.at[pl.ds(sid * RPT, RPT)],
                    out_hbm.at[cid, pl.ds(sid * RPT, RPT)])


def _deg_call(dst3, zeros16, ones16, interpret=False):
    f = pl.kernel(
        _deg_body,
        out_type=jax.ShapeDtypeStruct((NC, NP), jnp.float32),
        mesh=_sc_mesh(),
        scratch_types=[
            pltpu.VMEM((CH, K), jnp.int32),
            pltpu.VMEM((K,), jnp.float32),
            pltpu.VMEM_SHARED((NP,), jnp.float32),
            pltpu.SemaphoreType.DMA,
        ],
        interpret=interpret,
    )
    return f(dst3, zeros16, ones16)


def _prop_body(sd_hbm, xws_hbm, zeros_hbm, out_hbm,
               idx_v, rows0, acc_sp, gsem0):
    cid = lax.axis_index("c")
    sid = lax.axis_index("s")
    wid = cid * NS + sid
    pltpu.sync_copy(zeros_hbm, acc_sp.at[pl.ds(sid * RPT, RPT)])
    pltpu.sync_copy(sd_hbm.at[wid], idx_v)
    plsc.subcore_barrier()

    def chunk(i, carry):
        pltpu.async_copy(xws_hbm.at[idx_v.at[i]], rows0, gsem0).wait()
        pltpu.sync_copy(rows0, acc_sp.at[idx_v.at[CH + i]], add=True)
        return carry

    lax.fori_loop(0, CH, chunk, 0)
    plsc.subcore_barrier()
    pltpu.sync_copy(acc_sp.at[pl.ds(sid * RPT, RPT)],
                    out_hbm.at[cid, pl.ds(sid * RPT, RPT)])


def _prop_call(sd4, xws, zerosD, interpret=False):
    f = pl.kernel(
        _prop_body,
        out_type=jax.ShapeDtypeStruct((NC, NP, D), jnp.float32),
        mesh=_sc_mesh(),
        scratch_types=[
            pltpu.VMEM((2 * CH, K), jnp.int32),
            pltpu.VMEM((K, D), jnp.float32),
            pltpu.VMEM_SHARED((NP, D), jnp.float32),
            pltpu.SemaphoreType.DMA,
        ],
        interpret=interpret,
    )
    return f(sd4, xws, zerosD)


def _edge_body(src_hbm, dst_hbm, a_hbm, b_hbm, c_hbm, w2_hbm, out_hbm,
               src_v, dst_v, a0, b0, c0, z0, a1, b1, c1, z1, w2_v,
               gsem0, gsem1, zsem0, zsem1):
    cid = lax.axis_index("c")
    sid = lax.axis_index("s")
    wid = cid * NS + sid
    pltpu.sync_copy(src_hbm.at[wid], src_v)
    pltpu.sync_copy(dst_hbm.at[wid], dst_v)
    pltpu.sync_copy(w2_hbm, w2_v)
    w2s = [w2_v[pl.ds(16 * d, 16)] for d in range(D // 16)]
    bufs = ((a0, b0, c0, z0, gsem0, zsem0), (a1, b1, c1, z1, gsem1, zsem1))

    def issue(i, a_v, b_v, c_v, gsem):
        pltpu.async_copy(a_hbm.at[src_v.at[i]], a_v, gsem)
        pltpu.async_copy(b_hbm.at[dst_v.at[i]], b_v, gsem)
        pltpu.async_copy(c_hbm.at[wid, i], c_v, gsem)

    issue(0, a0, b0, c0, gsem0)
    issue(1, a1, b1, c1, gsem1)

    def pair(p, carry):
        for bsel in range(2):
            a_v, b_v, c_v, z_v, gsem, zsem = bufs[bsel]
            i = 2 * p + bsel
            for dbuf in (a_v, b_v, c_v):
                pltpu.make_async_copy(c_hbm.at[wid, 0], dbuf, gsem).wait()
            @pl.when(p > 0)
            def _():
                pltpu.make_async_copy(z_v, out_hbm.at[wid, 0], zsem).wait()

            @plsc.parallel_loop(0, KE, unroll=4)
            def edge_fn(e):
                s_acc = jnp.zeros((L,), jnp.float32)
                for d in range(D // 16):
                    va = a_v[e, pl.ds(16 * d, 16)]
                    vb = b_v[e, pl.ds(16 * d, 16)]
                    vc = c_v[e, pl.ds(16 * d, 16)]
                    s = jnp.maximum(va + vb + vc, 0.0)
                    s_acc = s_acc + s * w2s[d]
                z_v[e, :] = s_acc

            pltpu.async_copy(z_v, out_hbm.at[wid, i], zsem)

            @pl.when(i + 2 < CHE)
            def _():
                issue(i + 2, a_v, b_v, c_v, gsem)
        return carry

    lax.fori_loop(0, CHE // 2, pair, 0)
    for a_v, b_v, c_v, z_v, gsem, zsem in bufs:
        pltpu.make_async_copy(z_v, out_hbm.at[wid, 0], zsem).wait()


def _edge_call(src3, dst3, A, B, C4, w2, interpret=False):
    f = pl.kernel(
        _edge_body,
        out_type=jax.ShapeDtypeStruct((NW, CHE, KE, L), jnp.float32),
        mesh=_sc_mesh(),
        scratch_types=[
            pltpu.VMEM((CHE, KE), jnp.int32),
            pltpu.VMEM((CHE, KE), jnp.int32),
            pltpu.VMEM((KE, D), jnp.float32),
            pltpu.VMEM((KE, D), jnp.float32),
            pltpu.VMEM((KE, D), jnp.float32),
            pltpu.VMEM((KE, L), jnp.float32),
            pltpu.VMEM((KE, D), jnp.float32),
            pltpu.VMEM((KE, D), jnp.float32),
            pltpu.VMEM((KE, D), jnp.float32),
            pltpu.VMEM((KE, L), jnp.float32),
            pltpu.VMEM((D,), jnp.float32),
            pltpu.SemaphoreType.DMA,
            pltpu.SemaphoreType.DMA,
            pltpu.SemaphoreType.DMA,
            pltpu.SemaphoreType.DMA,
        ],
        interpret=interpret,
    )
    return f(src3, dst3, A, B, C4, w2)


def _fin_body(z_ref, sel_ref, b2_ref, out_ref):
    out_ref[...] = lax.dot_general(
        z_ref[...], sel_ref[...], (((1,), (0,)), ((), ())),
        preferred_element_type=jnp.float32) + b2_ref[0]


def _fin_call(z2, sel, b2, interpret=False):
    f = pl.pallas_call(
        _fin_body,
        out_shape=jax.ShapeDtypeStruct((E // 32, 32), jnp.float32),
        interpret=interpret,
    )
    return f(z2, sel, b2)


def _lstm_gates(wt, h, W_ih, W_hh, b_ih, b_hh):
    dn = (((1,), (1,)), ((), ()))
    gates = (lax.dot_general(wt, W_ih, dn, preferred_element_type=jnp.float32)
             + lax.dot_general(h, W_hh, dn, preferred_element_type=jnp.float32)
             + b_ih[None, :] + b_hh[None, :])
    return gates


def _prep_body(x_ref, w0t_ref, wih_ref, whh_ref, bih_ref, bhh_ref, degp_ref,
               xw_ref, xws_ref, dis_ref):
    wt = w0t_ref[...]
    h = jnp.zeros((D, D), jnp.float32)
    c = jnp.zeros((D, D), jnp.float32)
    W_ih = wih_ref[...]
    W_hh = whh_ref[...]
    b_ih = bih_ref[...]
    b_hh = bhh_ref[...]
    for _ in range(T_STEPS):
        gates = _lstm_gates(wt, h, W_ih, W_hh, b_ih, b_hh)
        gi = jax.nn.sigmoid(gates[:, 0 * D:1 * D])
        gf = jax.nn.sigmoid(gates[:, 1 * D:2 * D])
        gg = jnp.tanh(gates[:, 2 * D:3 * D])
        go = jax.nn.sigmoid(gates[:, 3 * D:4 * D])
        c = gf * c + gi * gg
        h = go * jnp.tanh(c)
        wt = h
    xw = lax.dot_general(x_ref[...], h, (((1,), (1,)), ((), ())),
                         preferred_element_type=jnp.float32)
    deg = degp_ref[0, :N] + degp_ref[1, :N] + 1.0
    dis = lax.rsqrt(deg)
    xw_ref[...] = xw
    xws_ref[...] = xw * dis[:, None]
    dis_ref[...] = dis


def _prep_call(x, w0t, wih, whh, bih, bhh, degp, interpret=False):
    f = pl.pallas_call(
        _prep_body,
        out_shape=[
            jax.ShapeDtypeStruct((N, D), jnp.float32),
            jax.ShapeDtypeStruct((N, D), jnp.float32),
            jax.ShapeDtypeStruct((N,), jnp.float32),
        ],
        interpret=interpret,
    )
    return f(x, w0t, wih, whh, bih, bhh, degp)


def _node_body(pacc_ref, xw_ref, dis_ref, w1a_ref, w1b_ref, b1_ref,
               a_ref, b_ref):
    dis = dis_ref[...]
    s = ((pacc_ref[0, :N] + pacc_ref[1, :N]) * dis[:, None]
         + (dis * dis)[:, None] * xw_ref[...])
    emb = jnp.maximum(s, 0.0)
    dn = (((1,), (1,)), ((), ()))
    a_ref[...] = lax.dot_general(emb, w1a_ref[...], dn,
                                 preferred_element_type=jnp.float32)
    b_ref[...] = lax.dot_general(emb, w1b_ref[...], dn,
                                 preferred_element_type=jnp.float32) + b1_ref[...][None, :]


def _node_call(pacc, xw, dis, w1a, w1b, b1, interpret=False):
    f = pl.pallas_call(
        _node_body,
        out_shape=[
            jax.ShapeDtypeStruct((N, D), jnp.float32),
            jax.ShapeDtypeStruct((N, D), jnp.float32),
        ],
        interpret=interpret,
    )
    return f(pacc, xw, dis, w1a, w1b, b1)


def _edgec_body(attr_ref, w1c_ref, c_ref):
    c_ref[...] = lax.dot_general(attr_ref[...], w1c_ref[...],
                                 (((1,), (1,)), ((), ())),
                                 preferred_element_type=jnp.float32)


def _edgec_call(edge_attr, w1c, interpret=False):
    f = pl.pallas_call(
        _edgec_body,
        grid=(E // EB,),
        in_specs=[
            pl.BlockSpec((EB, DE), lambda i: (i, 0)),
            pl.BlockSpec((D, DE), lambda i: (0, 0)),
        ],
        out_specs=pl.BlockSpec((EB, D), lambda i: (i, 0)),
        out_shape=jax.ShapeDtypeStruct((E, D), jnp.float32),
        interpret=interpret,
    )
    return f(edge_attr, w1c)


def _run(x, edge_index, edge_attr, initial_weights, lstm_W_ih, lstm_W_hh,
         lstm_b_ih, lstm_b_hh, mlp_W1, mlp_b1, mlp_W2, mlp_b2,
         interpret=False):
    src = edge_index[0]
    dst = edge_index[1]
    src3 = src.reshape(NW, CH, K)
    dst3 = dst.reshape(NW, CH, K)

    zeros1 = jnp.zeros((RPT,), jnp.float32)
    ones1 = jnp.ones((K,), jnp.float32)
    degp = _deg_call(dst3, zeros1, ones1, interpret=interpret)

    w1a = mlp_W1[:, :D]
    w1b = mlp_W1[:, D:2 * D]
    w1c = mlp_W1[:, 2 * D:]
    C = _edgec_call(edge_attr, w1c, interpret=interpret)
    C4 = C.reshape(NW, CHE, KE, D)

    xw, xws, dis = _prep_call(x, initial_weights.T, lstm_W_ih, lstm_W_hh,
                              lstm_b_ih, lstm_b_hh, degp, interpret=interpret)

    zerosD = jnp.zeros((RPT, D), jnp.float32)
    sd4 = jnp.concatenate([src3, dst3], axis=1)
    pacc = _prop_call(sd4, xws, zerosD, interpret=interpret)

    A, B = _node_call(pacc, xw, dis, w1a, w1b, mlp_b1, interpret=interpret)

    w2 = mlp_W2[0]
    srcE = src.reshape(NW, CHE, KE)
    dstE = dst.reshape(NW, CHE, KE)
    z = _edge_call(srcE, dstE, A, B, C4, w2, interpret=interpret)
    sel = jnp.repeat(jnp.eye(32, dtype=jnp.float32), L, axis=0)
    out2 = _fin_call(z.reshape(E // 32, 32 * L), sel, mlp_b2,
                     interpret=interpret)
    return out2.reshape(E)


def kernel(x, edge_index, edge_attr, initial_weights, lstm_W_ih, lstm_W_hh,
           lstm_b_ih, lstm_b_hh, mlp_W1, mlp_b1, mlp_W2, mlp_b2):
    return _run(x, edge_index, edge_attr, initial_weights, lstm_W_ih,
                lstm_W_hh, lstm_b_ih, lstm_b_hh, mlp_W1, mlp_b1, mlp_W2,
                mlp_b2)
